# baseline (speedup 1.0000x reference)
import jax
import jax.numpy as jnp
from jax import lax
from jax.experimental import pallas as pl
from jax.experimental.pallas import tpu as pltpu

_CIN, _COUT, _KH, _KW = 3, 8, 5, 5
_POOL = 64
_PH, _PW = 5, 7
_OH, _OW = _PH * _POOL, _PW * _POOL
_H, _W = _OH + _KH - 1, _OW + _KW - 1
_WPAD = 512
_FC_OUT = 10
_BN_EPS = 1e-5
_NG = _CIN * _KH
_KDIM = _NG * 128


def _fused_kernel(x_ref, wga_ref, wgb_ref, sha_ref, shb_ref,
                  fcw_ref, fcb_ref, out_ref, xb_ref, xs_ref, ba_ref, bb_ref):
    @pl.when(pl.program_id(0) == 0)
    def _build_b():
        def band(ncols, off, wg_ref):
            shp = (_KDIM, ncols)
            lane = lax.broadcasted_iota(jnp.int32, shp, 1)
            m = lane // 256 * 32 + lane % 32
            delta = lax.broadcasted_iota(jnp.int32, shp, 0) % 128 - m - off
            acc = jnp.zeros(shp, jnp.bfloat16)
            for kw in range(_KW):
                wrow = jnp.broadcast_to(wg_ref[:, kw, :][:, None, :],
                                        (_NG, 128, ncols)).reshape(shp)
                acc = jnp.where(delta == kw, wrow, acc)
            return acc

        ba_ref[...] = band(768, 0, wga_ref)
        bb_ref[...] = band(256, 32, wgb_ref)

    xb_ref[:, :, _W:] = jnp.zeros((_CIN, _H, _WPAD - _W), jnp.bfloat16)
    xb_ref[:, :, :_W] = x_ref[...].astype(jnp.bfloat16)
    xs_ref[...] = xb_ref[:, :, 64:64 + 384]

    def xcat(src_ref, j):
        slabs = []
        for ci in range(_CIN):
            for kh in range(_KH):
                slabs.append(src_ref[ci, kh:kh + _OH, 128 * j:128 * j + 128])
        return jnp.concatenate(slabs, axis=1)

    def pool32(r, shift, ncols):
        cb = jnp.max(r.reshape(_PH, _POOL, ncols), axis=1)
        p = jnp.max(cb.reshape(_PH, ncols // 32, 32), axis=2)
        return jnp.maximum(p + shift, 0.0)

    def adot(j, nblk):
        xc = xcat(xb_ref, j)
        r = lax.dot_general(xc, ba_ref[:, :256 * nblk],
                            (((1,), (0,)), ((), ())),
                            preferred_element_type=jnp.float32)
        return (pool32(r, sha_ref[:, :8 * nblk], 256 * nblk)
                .reshape(_PH, nblk, _COUT))

    def bdot(j):
        xc = xcat(xs_ref, j)
        r = lax.dot_general(xc, bb_ref[...], (((1,), (0,)), ((), ())),
                            preferred_element_type=jnp.float32)
        return pool32(r, shb_ref[...], 256).reshape(_PH, 1, _COUT)

    def fc_part(pooled_j, pw0):
        w = fcw_ref[:, pw0:pw0 + pooled_j.shape[1], :, :]
        return jnp.sum(pooled_j[..., None] * w, axis=(0, 1, 2))[None, :]

    def pair(pa, pb):
        return jnp.concatenate(
            [jnp.maximum(pa[:, 0:1], pa[:, 1:2]),
             jnp.maximum(pa[:, 2:3], pb)], axis=1)

    acc = fcb_ref[...]
    acc = acc + fc_part(pair(adot(0, 3), bdot(0)), 0)
    acc = acc + fc_part(pair(adot(1, 3), bdot(1)), 2)
    a2 = adot(2, 3)
    a3 = adot(3, 2)
    acc = acc + fc_part(jnp.maximum(a3[:, 0:1], a3[:, 1:2]), 6)
    acc = acc + fc_part(pair(a2, bdot(2)), 4)
    out_ref[...] = acc


def _forward(x, conv_w, conv_b, bn_gamma, bn_beta, bn_mean, bn_var, fc_w, fc_b):
    n = x.shape[0]

    bn_scale = bn_gamma / jnp.sqrt(bn_var + _BN_EPS)
    wf = conv_w * bn_scale[:, None, None, None]
    bf = (conv_b - bn_mean) * bn_scale + bn_beta

    wg = (wf.transpose(1, 2, 3, 0).reshape(_NG, _KW, _COUT)
          .astype(jnp.bfloat16))
    wgb = jnp.repeat(wg, 32, axis=2)
    wga = jnp.tile(wgb, (1, 1, 3))

    sha = jnp.tile(bf, 3).reshape(1, _COUT * 3)
    shb = bf.reshape(1, _COUT)

    fcw_r = fc_w.reshape(_FC_OUT, _COUT, _PH, _PW).transpose(2, 3, 1, 0)
    fcb2 = fc_b.reshape(1, _FC_OUT)

    flops = 2 * n * 320 * _KDIM * (3 * 768 + 512 + 3 * 256)
    bytes_accessed = (4 * n * _CIN * _H * _W + 2 * _KDIM * 1024
                      + 4 * n * _FC_OUT)

    out = pl.pallas_call(
        _fused_kernel,
        out_shape=jax.ShapeDtypeStruct((n, 1, _FC_OUT), jnp.float32),
        grid_spec=pltpu.PrefetchScalarGridSpec(
            num_scalar_prefetch=0,
            grid=(n,),
            in_specs=[
                pl.BlockSpec((None, _CIN, _H, _W), lambda i: (i, 0, 0, 0)),
                pl.BlockSpec((_NG, _KW, _COUT * 96), lambda i: (0, 0, 0)),
                pl.BlockSpec((_NG, _KW, _COUT * 32), lambda i: (0, 0, 0)),
                pl.BlockSpec((1, _COUT * 3), lambda i: (0, 0)),
                pl.BlockSpec((1, _COUT), lambda i: (0, 0)),
                pl.BlockSpec((_PH, _PW, _COUT, _FC_OUT), lambda i: (0, 0, 0, 0)),
                pl.BlockSpec((1, _FC_OUT), lambda i: (0, 0)),
            ],
            out_specs=pl.BlockSpec((None, 1, _FC_OUT), lambda i: (i, 0, 0)),
            scratch_shapes=[
                pltpu.VMEM((_CIN, _H, _WPAD), jnp.bfloat16),
                pltpu.VMEM((_CIN, _H, 384), jnp.bfloat16),
                pltpu.VMEM((_KDIM, _COUT * 96), jnp.bfloat16),
                pltpu.VMEM((_KDIM, _COUT * 32), jnp.bfloat16),
            ],
        ),
        compiler_params=pltpu.CompilerParams(
            dimension_semantics=("arbitrary",),
            vmem_limit_bytes=48 * 1024 * 1024),
        cost_estimate=pl.CostEstimate(
            flops=flops, transcendentals=0, bytes_accessed=bytes_accessed),
    )(x, wga, wgb, sha, shb, fcw_r, fcb2)
    return out.reshape(n, _FC_OUT)


def kernel(x, conv_w, conv_b, bn_gamma, bn_beta, bn_mean, bn_var, fc_w, fc_b):
    return _forward(x, conv_w, conv_b, bn_gamma, bn_beta, bn_mean,
                    bn_var, fc_w, fc_b)


# speedup vs baseline: 1.2925x; 1.2925x over previous
import jax
import jax.numpy as jnp
from jax import lax
from jax.experimental import pallas as pl
from jax.experimental.pallas import tpu as pltpu

_CIN, _COUT, _KH, _KW = 3, 8, 5, 5
_POOL = 64
_PH, _PW = 5, 7
_OH, _OW = _PH * _POOL, _PW * _POOL
_H, _W = _OH + _KH - 1, _OW + _KW - 1
_WPAD = 512
_FC_OUT = 10
_BN_EPS = 1e-5
_NG = _CIN * _KH
_KDIM = 1024


def _fused_kernel(x_ref, wgb_ref, shb_ref, fcw_ref, fcb_ref, out_ref,
                  xb_ref, xr64_ref, xl32_ref, xr32_ref, bq_ref):
    @pl.when(pl.program_id(0) == 0)
    def _build_b():
        shp = (_KDIM, 256)
        delta = (lax.broadcasted_iota(jnp.int32, shp, 0) % 64
                 - lax.broadcasted_iota(jnp.int32, shp, 1) % 32)
        acc = jnp.zeros(shp, jnp.bfloat16)
        for kw in range(_KW):
            w64 = jnp.concatenate(
                [jnp.broadcast_to(wgb_ref[:, kw, :][:, None, :],
                                  (_NG, 64, 256)).reshape(_NG * 64, 256),
                 jnp.zeros((64, 256), jnp.bfloat16)], axis=0)
            acc = jnp.where(delta == kw, w64, acc)
        bq_ref[...] = acc

    xb_ref[:, :, _W:] = jnp.zeros((_CIN, _H, _WPAD - _W), jnp.bfloat16)
    xb_ref[:, :, :_W] = x_ref[...].astype(jnp.bfloat16)
    xr64_ref[:, :, :64] = jnp.zeros((_CIN, _H, 64), jnp.bfloat16)
    xr64_ref[:, :, 64:] = xb_ref[:, :, :448]
    xl32_ref[:, :, 480:] = jnp.zeros((_CIN, _H, 32), jnp.bfloat16)
    xl32_ref[:, :, :480] = xb_ref[:, :, 32:]
    xr32_ref[:, :, :32] = jnp.zeros((_CIN, _H, 32), jnp.bfloat16)
    xr32_ref[:, :, 32:] = xb_ref[:, :, :480]

    def xcat(b):
        c0 = 32 * b
        pieces = []
        for q in range(_NG):
            e = q % 2
            if b % 2 == 0:
                if e == (c0 // 64) % 2:
                    src, a = xb_ref, c0
                else:
                    src, a = xr64_ref, c0 + 64
            else:
                m = (c0 - 32) // 64
                if e == m % 2:
                    src, a = xl32_ref, 64 * m
                else:
                    src, a = xr32_ref, 64 * m + 64
            ci, kh = q // _KH, q % _KH
            pieces.append(src[ci, kh:kh + _OH, a:a + 64])
        pieces.append(jnp.zeros((_OH, 64), jnp.bfloat16))
        return jnp.concatenate(pieces, axis=1)

    def bdot(b):
        r = lax.dot_general(xcat(b), bq_ref[...], (((1,), (0,)), ((), ())),
                            preferred_element_type=jnp.float32)
        cb = jnp.max(r.reshape(_PH, _POOL, 256), axis=1)
        p = jnp.max(cb.reshape(_PH, _COUT, 32), axis=2)
        return jnp.maximum(p + shb_ref[...], 0.0)

    acc = fcb_ref[...]
    for p in range(_PW):
        pooled = jnp.maximum(bdot(2 * p), bdot(2 * p + 1))
        w = fcw_ref[:, p, :, :]
        acc = acc + jnp.sum(pooled[..., None] * w, axis=(0, 1))[None, :]
    out_ref[...] = acc


def _forward(x, conv_w, conv_b, bn_gamma, bn_beta, bn_mean, bn_var, fc_w, fc_b):
    n = x.shape[0]

    bn_scale = bn_gamma / jnp.sqrt(bn_var + _BN_EPS)
    wf = conv_w * bn_scale[:, None, None, None]
    bf = (conv_b - bn_mean) * bn_scale + bn_beta

    wg = (wf.transpose(1, 2, 3, 0).reshape(_NG, _KW, _COUT)
          .astype(jnp.bfloat16))
    wgb = jnp.repeat(wg, 32, axis=2)

    shb = bf.reshape(1, _COUT)
    fcw_r = fc_w.reshape(_FC_OUT, _COUT, _PH, _PW).transpose(2, 3, 1, 0)
    fcb2 = fc_b.reshape(1, _FC_OUT)

    flops = 2 * n * 320 * _KDIM * 14 * 256
    bytes_accessed = (4 * n * _CIN * _H * _W + 2 * _KDIM * 256
                      + 4 * n * _FC_OUT)

    out = pl.pallas_call(
        _fused_kernel,
        out_shape=jax.ShapeDtypeStruct((n, 1, _FC_OUT), jnp.float32),
        grid_spec=pltpu.PrefetchScalarGridSpec(
            num_scalar_prefetch=0,
            grid=(n,),
            in_specs=[
                pl.BlockSpec((None, _CIN, _H, _W), lambda i: (i, 0, 0, 0)),
                pl.BlockSpec((_NG, _KW, 256), lambda i: (0, 0, 0)),
                pl.BlockSpec((1, _COUT), lambda i: (0, 0)),
                pl.BlockSpec((_PH, _PW, _COUT, _FC_OUT), lambda i: (0, 0, 0, 0)),
                pl.BlockSpec((1, _FC_OUT), lambda i: (0, 0)),
            ],
            out_specs=pl.BlockSpec((None, 1, _FC_OUT), lambda i: (i, 0, 0)),
            scratch_shapes=[
                pltpu.VMEM((_CIN, _H, _WPAD), jnp.bfloat16),
                pltpu.VMEM((_CIN, _H, _WPAD), jnp.bfloat16),
                pltpu.VMEM((_CIN, _H, _WPAD), jnp.bfloat16),
                pltpu.VMEM((_CIN, _H, _WPAD), jnp.bfloat16),
                pltpu.VMEM((_KDIM, 256), jnp.bfloat16),
            ],
        ),
        compiler_params=pltpu.CompilerParams(
            dimension_semantics=("arbitrary",),
            vmem_limit_bytes=48 * 1024 * 1024),
        cost_estimate=pl.CostEstimate(
            flops=flops, transcendentals=0, bytes_accessed=bytes_accessed),
    )(x, wgb, shb, fcw_r, fcb2)
    return out.reshape(n, _FC_OUT)


def kernel(x, conv_w, conv_b, bn_gamma, bn_beta, bn_mean, bn_var, fc_w, fc_b):
    return _forward(x, conv_w, conv_b, bn_gamma, bn_beta, bn_mean,
                    bn_var, fc_w, fc_b)
